# baseline (device time: 9910 ns/iter reference)
import jax
import jax.numpy as jnp
from jax import lax
from jax.experimental import pallas as pl
from jax.experimental.pallas import tpu as pltpu

N_DEV = 4


def kernel(x):
    m, n_total = x.shape
    n = n_total // N_DEV

    def body(x_hbm, out_hbm, xv_ref, sb_ref, in_sem, out_sem,
             send_sems, recv_sems):
        my = lax.axis_index("i")

        copy_in = pltpu.make_async_copy(x_hbm, xv_ref, in_sem)
        copy_in.start()

        barrier_sem = pltpu.get_barrier_semaphore()
        for d in range(1, N_DEV):
            peer = (my + d) % N_DEV
            pl.semaphore_signal(
                barrier_sem, inc=1,
                device_id=(peer,), device_id_type=pl.DeviceIdType.MESH,
            )

        copy_in.wait()

        for d in range(1, N_DEV):
            dst = (my + d) % N_DEV
            sb_ref[d - 1, :, :] = xv_ref[:, pl.ds(dst * n, n)].astype(
                jnp.bfloat16
            )
        sb_ref[N_DEV - 1, :, :] = xv_ref[:, pl.ds(my * n, n)].astype(
            jnp.bfloat16
        )

        copy_out = pltpu.make_async_copy(
            sb_ref.at[N_DEV - 1], out_hbm.at[pl.ds(my * m, m), :], out_sem
        )
        copy_out.start()

        pl.semaphore_wait(barrier_sem, N_DEV - 1)

        sends = []
        for d in (2, 1, 3):
            dst = (my + d) % N_DEV
            rdma = pltpu.make_async_remote_copy(
                src_ref=sb_ref.at[d - 1],
                dst_ref=out_hbm.at[pl.ds(my * m, m), :],
                send_sem=send_sems.at[d - 1],
                recv_sem=recv_sems.at[my],
                device_id=(dst,),
                device_id_type=pl.DeviceIdType.MESH,
            )
            rdma.start()
            sends.append(rdma)

        for d in range(1, N_DEV):
            src = (my - d) % N_DEV
            recv = pltpu.make_async_remote_copy(
                src_ref=sb_ref.at[d - 1],
                dst_ref=out_hbm.at[pl.ds(src * m, m), :],
                send_sem=send_sems.at[d - 1],
                recv_sem=recv_sems.at[src],
                device_id=(src,),
                device_id_type=pl.DeviceIdType.MESH,
            )
            recv.wait_recv()

        for rdma in sends:
            rdma.wait_send()
        copy_out.wait()

    out_shape = jax.ShapeDtypeStruct((N_DEV * m, n), jnp.bfloat16)
    return pl.pallas_call(
        body,
        out_shape=out_shape,
        in_specs=[pl.BlockSpec(memory_space=pl.ANY)],
        out_specs=pl.BlockSpec(memory_space=pl.ANY),
        scratch_shapes=[
            pltpu.VMEM((m, n_total), jnp.float32),
            pltpu.VMEM((N_DEV, m, n), jnp.bfloat16),
            pltpu.SemaphoreType.DMA,
            pltpu.SemaphoreType.DMA,
            pltpu.SemaphoreType.DMA((N_DEV - 1,)),
            pltpu.SemaphoreType.DMA((N_DEV,)),
        ],
        compiler_params=pltpu.CompilerParams(collective_id=0),
    )(x)


# device time: 9840 ns/iter; 1.0071x vs baseline; 1.0071x over previous
import jax
import jax.numpy as jnp
from jax import lax
from jax.experimental import pallas as pl
from jax.experimental.pallas import tpu as pltpu

N_DEV = 4


def kernel(x):
    m, n_total = x.shape
    n = n_total // N_DEV

    def body(x_ref, out_ref, sb_ref, ready_sems, send_sems, recv_sems):
        my = lax.axis_index("i")

        barrier_sem = pltpu.get_barrier_semaphore()
        pl.semaphore_signal(barrier_sem, inc=1)

        for d in range(1, N_DEV):
            peer = (my + d) % N_DEV
            pl.semaphore_signal(
                ready_sems.at[my], inc=1,
                device_id=(peer,), device_id_type=pl.DeviceIdType.MESH,
            )

        for d in range(1, N_DEV):
            dst = (my + d) % N_DEV
            sb_ref[d - 1, :, :] = x_ref[:, pl.ds(dst * n, n)].astype(
                jnp.bfloat16
            )
        out_ref[pl.ds(my * m, m), :] = x_ref[:, pl.ds(my * n, n)].astype(
            jnp.bfloat16
        )

        sends = []
        for d in (1, 3, 2):
            dst = (my + d) % N_DEV
            pl.semaphore_wait(ready_sems.at[dst], 1)
            rdma = pltpu.make_async_remote_copy(
                src_ref=sb_ref.at[d - 1],
                dst_ref=out_ref.at[pl.ds(my * m, m), :],
                send_sem=send_sems.at[d - 1],
                recv_sem=recv_sems.at[my],
                device_id=(dst,),
                device_id_type=pl.DeviceIdType.MESH,
            )
            rdma.start()
            sends.append(rdma)

        pl.semaphore_wait(barrier_sem, 1)

        for d in range(1, N_DEV):
            src = (my - d) % N_DEV
            recv = pltpu.make_async_remote_copy(
                src_ref=sb_ref.at[d - 1],
                dst_ref=out_ref.at[pl.ds(src * m, m), :],
                send_sem=send_sems.at[d - 1],
                recv_sem=recv_sems.at[src],
                device_id=(src,),
                device_id_type=pl.DeviceIdType.MESH,
            )
            recv.wait_recv()

        for rdma in sends:
            rdma.wait_send()

    out_shape = jax.ShapeDtypeStruct((N_DEV * m, n), jnp.bfloat16)
    return pl.pallas_call(
        body,
        out_shape=out_shape,
        in_specs=[pl.BlockSpec(memory_space=pltpu.VMEM)],
        out_specs=pl.BlockSpec(memory_space=pltpu.VMEM),
        scratch_shapes=[
            pltpu.VMEM((N_DEV - 1, m, n), jnp.bfloat16),
            pltpu.SemaphoreType.REGULAR((N_DEV,)),
            pltpu.SemaphoreType.DMA((N_DEV - 1,)),
            pltpu.SemaphoreType.DMA((N_DEV,)),
        ],
        compiler_params=pltpu.CompilerParams(collective_id=0),
    )(x)


# device time: 8617 ns/iter; 1.1501x vs baseline; 1.1419x over previous
import jax
import jax.numpy as jnp
from jax import lax
from jax.experimental import pallas as pl
from jax.experimental.pallas import tpu as pltpu

N_DEV = 4


def kernel(x):
    m, n_total = x.shape
    n = n_total // N_DEV

    def body(x_ref, out_ref, sb_ref, ready_sems, send_sems, recv_sems):
        my = lax.axis_index("i")

        barrier_sem = pltpu.get_barrier_semaphore()
        pl.semaphore_signal(barrier_sem, inc=1)

        for d in range(1, N_DEV):
            peer = (my + d) % N_DEV
            pl.semaphore_signal(
                ready_sems.at[my], inc=1,
                device_id=(peer,), device_id_type=pl.DeviceIdType.MESH,
            )

        for d in range(1, N_DEV):
            dst = (my + d) % N_DEV
            sb_ref[d - 1, :, :] = x_ref[:, pl.ds(dst * n, n)].astype(
                jnp.bfloat16
            )
        out_ref[pl.ds(my * m, m), :] = x_ref[:, pl.ds(my * n, n)].astype(
            jnp.bfloat16
        )

        send_orders = {0: (1, 3, 2), 1: (3, 2, 1), 2: (2, 3, 1), 3: (1, 2, 3)}
        for k in range(N_DEV):

            @pl.when(my == k)
            def _(k=k):
                for d in send_orders[k]:
                    dst = (k + d) % N_DEV
                    pl.semaphore_wait(ready_sems.at[dst], 1)
                    rdma = pltpu.make_async_remote_copy(
                        src_ref=sb_ref.at[d - 1],
                        dst_ref=out_ref.at[pl.ds(k * m, m), :],
                        send_sem=send_sems.at[d - 1],
                        recv_sem=recv_sems.at[k],
                        device_id=(dst,),
                        device_id_type=pl.DeviceIdType.MESH,
                    )
                    rdma.start()

        pl.semaphore_wait(barrier_sem, 1)

        for d in range(1, N_DEV):
            src = (my - d) % N_DEV
            recv = pltpu.make_async_remote_copy(
                src_ref=sb_ref.at[d - 1],
                dst_ref=out_ref.at[pl.ds(src * m, m), :],
                send_sem=send_sems.at[d - 1],
                recv_sem=recv_sems.at[src],
                device_id=(src,),
                device_id_type=pl.DeviceIdType.MESH,
            )
            recv.wait_recv()

        for d in range(1, N_DEV):
            drain = pltpu.make_async_remote_copy(
                src_ref=sb_ref.at[d - 1],
                dst_ref=out_ref.at[pl.ds(my * m, m), :],
                send_sem=send_sems.at[d - 1],
                recv_sem=recv_sems.at[my],
                device_id=((my + d) % N_DEV,),
                device_id_type=pl.DeviceIdType.MESH,
            )
            drain.wait_send()

    out_shape = jax.ShapeDtypeStruct((N_DEV * m, n), jnp.bfloat16)
    return pl.pallas_call(
        body,
        out_shape=out_shape,
        in_specs=[pl.BlockSpec(memory_space=pltpu.VMEM)],
        out_specs=pl.BlockSpec(memory_space=pltpu.VMEM),
        scratch_shapes=[
            pltpu.VMEM((N_DEV - 1, m, n), jnp.bfloat16),
            pltpu.SemaphoreType.REGULAR((N_DEV,)),
            pltpu.SemaphoreType.DMA((N_DEV - 1,)),
            pltpu.SemaphoreType.DMA((N_DEV,)),
        ],
        compiler_params=pltpu.CompilerParams(collective_id=0),
    )(x)


# device time: 8583 ns/iter; 1.1546x vs baseline; 1.0040x over previous
import jax
import jax.numpy as jnp
from jax import lax
from jax.experimental import pallas as pl
from jax.experimental.pallas import tpu as pltpu

N_DEV = 4


def kernel(x):
    m, n_total = x.shape
    n = n_total // N_DEV

    def body(x_ref, out_ref, sb_ref, ready_sems, send_sems, recv_sems):
        my = lax.axis_index("i")

        barrier_sem = pltpu.get_barrier_semaphore()
        pl.semaphore_signal(barrier_sem, inc=1)

        for d in range(1, N_DEV):
            peer = (my + d) % N_DEV
            pl.semaphore_signal(
                ready_sems.at[my], inc=1,
                device_id=(peer,), device_id_type=pl.DeviceIdType.MESH,
            )

        send_orders = {0: (1, 3, 2), 1: (3, 2, 1), 2: (2, 3, 1), 3: (1, 2, 3)}
        for k in range(N_DEV):

            @pl.when(my == k)
            def _(k=k):
                for d in send_orders[k]:
                    dst = (k + d) % N_DEV
                    sb_ref[d - 1, :, :] = x_ref[:, pl.ds(dst * n, n)].astype(
                        jnp.bfloat16
                    )
                    pl.semaphore_wait(ready_sems.at[dst], 1)
                    rdma = pltpu.make_async_remote_copy(
                        src_ref=sb_ref.at[d - 1],
                        dst_ref=out_ref.at[pl.ds(k * m, m), :],
                        send_sem=send_sems.at[d - 1],
                        recv_sem=recv_sems.at[k],
                        device_id=(dst,),
                        device_id_type=pl.DeviceIdType.MESH,
                    )
                    rdma.start()

        out_ref[pl.ds(my * m, m), :] = x_ref[:, pl.ds(my * n, n)].astype(
            jnp.bfloat16
        )

        pl.semaphore_wait(barrier_sem, 1)

        for d in range(1, N_DEV):
            src = (my - d) % N_DEV
            recv = pltpu.make_async_remote_copy(
                src_ref=sb_ref.at[d - 1],
                dst_ref=out_ref.at[pl.ds(src * m, m), :],
                send_sem=send_sems.at[d - 1],
                recv_sem=recv_sems.at[src],
                device_id=(src,),
                device_id_type=pl.DeviceIdType.MESH,
            )
            recv.wait_recv()

        for d in range(1, N_DEV):
            drain = pltpu.make_async_remote_copy(
                src_ref=sb_ref.at[d - 1],
                dst_ref=out_ref.at[pl.ds(my * m, m), :],
                send_sem=send_sems.at[d - 1],
                recv_sem=recv_sems.at[my],
                device_id=((my + d) % N_DEV,),
                device_id_type=pl.DeviceIdType.MESH,
            )
            drain.wait_send()

    out_shape = jax.ShapeDtypeStruct((N_DEV * m, n), jnp.bfloat16)
    return pl.pallas_call(
        body,
        out_shape=out_shape,
        in_specs=[pl.BlockSpec(memory_space=pltpu.VMEM)],
        out_specs=pl.BlockSpec(memory_space=pltpu.VMEM),
        scratch_shapes=[
            pltpu.VMEM((N_DEV - 1, m, n), jnp.bfloat16),
            pltpu.SemaphoreType.REGULAR((N_DEV,)),
            pltpu.SemaphoreType.DMA((N_DEV - 1,)),
            pltpu.SemaphoreType.DMA((N_DEV,)),
        ],
        compiler_params=pltpu.CompilerParams(collective_id=0),
    )(x)
